# baseline (speedup 1.0000x reference)
import functools

import jax
import jax.numpy as jnp
from jax import lax
from jax.experimental import pallas as pl
from jax.experimental.pallas import tpu as pltpu
from jax.experimental.pallas import tpu_sc as plsc

D = 32
B = 16384
NC = 2
NS = 16
NW = NC * NS
BPW = B // NW
CHUNK = 128
NCHUNK = BPW // CHUNK
L = 16


def _body(u_hbm, v_hbm, uidx_hbm, iidx_hbm, out_hbm,
          uidx_v, iidx_v, urows_v, vrows_v, out_v, sem_u, sem_v):
    wid = lax.axis_index("s") * NC + lax.axis_index("c")
    base = wid * BPW

    for j in range(NCHUNK):
        pltpu.sync_copy(uidx_hbm.at[pl.ds(base + j * CHUNK, CHUNK)],
                        uidx_v.at[j])
        pltpu.sync_copy(iidx_hbm.at[pl.ds(base + j * CHUNK, CHUNK)],
                        iidx_v.at[j])

    cps_u = []
    cps_v = []
    for j in range(NCHUNK):
        cps_u.append(pltpu.async_copy(
            u_hbm.at[uidx_v.at[j]],
            urows_v.at[pl.ds(j * CHUNK, CHUNK)], sem_u))
        cps_v.append(pltpu.async_copy(
            v_hbm.at[iidx_v.at[j]],
            vrows_v.at[pl.ds(j * CHUNK, CHUNK)], sem_v))

    iota = lax.iota(jnp.int32, L)
    cols = [jnp.bitwise_and(iota + d, D - 1) for d in range(D)]

    def block_body(blk, carry):
        rows = iota + blk * L
        acc = jnp.zeros((L,), jnp.float32)
        for d in range(D):
            cu = plsc.load_gather(urows_v, [rows, cols[d]])
            cv = plsc.load_gather(vrows_v, [rows, cols[d]])
            acc = acc + cu * cv
        out_v[pl.ds(blk * L, L)] = acc
        return carry

    blocks_per_chunk = CHUNK // L
    for j in range(NCHUNK):
        cps_u[j].wait()
        cps_v[j].wait()
        lax.fori_loop(j * blocks_per_chunk, (j + 1) * blocks_per_chunk,
                      block_body, 0)

    pltpu.sync_copy(out_v, out_hbm.at[pl.ds(base, BPW)])


@jax.jit
def _run(U, V, user_idx, item_idx):
    mesh = plsc.VectorSubcoreMesh(core_axis_name="c", subcore_axis_name="s")
    f = functools.partial(
        pl.kernel,
        out_type=jax.ShapeDtypeStruct((B,), jnp.float32),
        mesh=mesh,
        compiler_params=pltpu.CompilerParams(
            use_tc_tiling_on_sc=False,
            needs_layout_passes=False,
        ),
        scratch_types=[
            pltpu.VMEM((NCHUNK, CHUNK), jnp.int32),
            pltpu.VMEM((NCHUNK, CHUNK), jnp.int32),
            pltpu.VMEM((BPW, D), jnp.float32),
            pltpu.VMEM((BPW, D), jnp.float32),
            pltpu.VMEM((BPW,), jnp.float32),
            pltpu.SemaphoreType.DMA,
            pltpu.SemaphoreType.DMA,
        ],
    )(_body)
    return f(U, V, user_idx, item_idx)


def kernel(U, V, user_idx, item_idx):
    return _run(U, V, user_idx.astype(jnp.int32), item_idx.astype(jnp.int32))


# speedup vs baseline: 2.3367x; 2.3367x over previous
import functools

import jax
import jax.numpy as jnp
from jax import lax
from jax.experimental import pallas as pl
from jax.experimental.pallas import tpu as pltpu
from jax.experimental.pallas import tpu_sc as plsc

D = 32
B = 16384
NC = 2
NS = 16
NW = NC * NS
BPW = B // NW
BATCH = 4
NBATCH = BPW // BATCH
L = 16


def _fire(ut3, vt3, uring, vring, sem_u, sem_v, uvec, ivec, voff, slot_base):
    for j in range(BATCH):
        u = uvec[voff + j]
        v = ivec[voff + j]
        cu = jax.lax.shift_right_logical(u, 7)
        cv = jax.lax.shift_right_logical(v, 7)
        pltpu.async_copy(
            ut3.at[:, :, pl.ds(cu * 128, 128)],
            uring.at[slot_base + j], sem_u)
        pltpu.async_copy(
            vt3.at[:, :, pl.ds(cv * 128, 128)],
            vring.at[slot_base + j], sem_v)


def _drain(ut3, uring, vring, sem_u, sem_v, slot_base):
    for j in range(BATCH):
        pltpu.make_async_copy(
            ut3.at[:, :, pl.ds(0, 128)], uring.at[slot_base + j], sem_u
        ).wait()
        pltpu.make_async_copy(
            ut3.at[:, :, pl.ds(0, 128)], vring.at[slot_base + j], sem_v
        ).wait()


def _process(uring, vring, lanes_u, lanes_v, out_v, obase, slot_base, consts):
    iota, q4, fold_v = consts
    opos = obase + (iota & 3)
    lu = plsc.load_gather(lanes_u, [opos])
    lv = plsc.load_gather(lanes_v, [opos])
    slotv = slot_base + (iota & 3)
    acc = jnp.zeros((L,), jnp.float32)
    for t in range(D // 4):
        d0 = 4 * t
        rv = jnp.full((L,), d0 // 8, jnp.int32)
        sv = (d0 % 8) + q4
        cu = plsc.load_gather(uring, [slotv, rv, sv, lu])
        cv = plsc.load_gather(vring, [slotv, rv, sv, lv])
        acc = acc + cu * cv
    fold_v[...] = acc
    h = (plsc.load_gather(fold_v, [iota & 7])
         + plsc.load_gather(fold_v, [(iota & 7) + 8]))
    fold_v[...] = h
    out4 = (plsc.load_gather(fold_v, [iota & 3])
            + plsc.load_gather(fold_v, [(iota & 3) + 4]))
    plsc.store_scatter(out_v, [opos], out4, mask=iota < BATCH)


def _body(ut3, vt3, uidx_hbm, iidx_hbm, out_hbm,
          uring, vring, uidx_v, iidx_v, lanes_u, lanes_v, out_v, fold_v,
          sem_u0, sem_v0, sem_u1, sem_v1):
    wid = lax.axis_index("s") * NC + lax.axis_index("c")
    base = wid * BPW

    pltpu.sync_copy(uidx_hbm.at[pl.ds(base, BPW)], uidx_v)
    pltpu.sync_copy(iidx_hbm.at[pl.ds(base, BPW)], iidx_v)

    iota = lax.iota(jnp.int32, L)
    q4 = jax.lax.shift_right_logical(iota, 2)
    consts = (iota, q4, fold_v)
    for k in range(BPW // L):
        lanes_u[pl.ds(k * L, L)] = uidx_v[pl.ds(k * L, L)] & 127
        lanes_v[pl.ds(k * L, L)] = iidx_v[pl.ds(k * L, L)] & 127

    fire = functools.partial(_fire, ut3, vt3, uring, vring)
    drain = functools.partial(_drain, ut3, uring, vring)
    proc = functools.partial(
        _process, uring, vring, lanes_u, lanes_v, out_v, consts=consts)

    sems = ((sem_u0, sem_v0), (sem_u1, sem_v1))
    pvec_u = uidx_v[pl.ds(0, L)]
    pvec_i = iidx_v[pl.ds(0, L)]
    fire(*sems[0], pvec_u, pvec_i, 0, 0)

    def step(i, carry):
        base16 = i * L
        uvec = uidx_v[pl.ds(base16, L)]
        ivec = iidx_v[pl.ds(base16, L)]
        nbase = jnp.minimum(base16 + L, BPW - L)
        nuvec = uidx_v[pl.ds(nbase, L)]
        nivec = iidx_v[pl.ds(nbase, L)]
        for b in range(4):
            par = b % 2
            npar = (b + 1) % 2
            if b < 3:
                fire(*sems[npar], uvec, ivec, 4 * (b + 1), npar * BATCH)
            else:
                @pl.when(i < NBATCH // 4 - 1)
                def _():
                    fire(*sems[npar], nuvec, nivec, 0, npar * BATCH)
            drain(*sems[par], par * BATCH)
            proc(base16 + 4 * b, par * BATCH)
        return carry

    lax.fori_loop(0, NBATCH // 4, step, 0)

    pltpu.sync_copy(out_v, out_hbm.at[pl.ds(base, BPW)])


@jax.jit
def _run(Ut3, Vt3, user_idx, item_idx):
    mesh = plsc.VectorSubcoreMesh(core_axis_name="c", subcore_axis_name="s")
    f = functools.partial(
        pl.kernel,
        out_type=jax.ShapeDtypeStruct((B,), jnp.float32),
        mesh=mesh,
        compiler_params=pltpu.CompilerParams(
            use_tc_tiling_on_sc=True,
            needs_layout_passes=False,
        ),
        scratch_types=[
            pltpu.VMEM((2 * BATCH, 4, 8, 128), jnp.float32),
            pltpu.VMEM((2 * BATCH, 4, 8, 128), jnp.float32),
            pltpu.VMEM((BPW,), jnp.int32),
            pltpu.VMEM((BPW,), jnp.int32),
            pltpu.VMEM((BPW,), jnp.int32),
            pltpu.VMEM((BPW,), jnp.int32),
            pltpu.VMEM((BPW,), jnp.float32),
            pltpu.VMEM((L,), jnp.float32),
            pltpu.SemaphoreType.DMA,
            pltpu.SemaphoreType.DMA,
            pltpu.SemaphoreType.DMA,
            pltpu.SemaphoreType.DMA,
        ],
    )(_body)
    return f(Ut3, Vt3, user_idx, item_idx)


def kernel(U, V, user_idx, item_idx):
    Ut3 = U.T.reshape(4, 8, U.shape[0])
    Vt3 = V.T.reshape(4, 8, V.shape[0])
    return _run(Ut3, Vt3,
                user_idx.astype(jnp.int32), item_idx.astype(jnp.int32))
